# speedup vs baseline: 1.6931x; 1.6931x over previous
import jax
import jax.numpy as jnp
from jax.experimental import pallas as pl
from jax.experimental.pallas import tpu as pltpu

B, S, D, C = 1024, 200, 128, 49
BB = 8


def _body(s_ref, e_ref, x_ref, w_ref, b_ref, out1_ref, out2_ref, xmod_ref):
    g = pl.program_id(0)
    iota = jax.lax.broadcasted_iota(jnp.int32, (S, 1), 0)
    for j in range(BB):
        i = g * BB + j
        s = s_ref[i]
        e = e_ref[i]
        xj = x_ref[j]
        vs = x_ref[j, pl.ds(s, 1), :]
        ve = x_ref[j, pl.ds(e, 1), :]
        v = 0.5 * (vs + ve)
        coef = ((iota == s) | (iota == e)).astype(jnp.float32)
        xmod_ref[pl.ds(j * S, S), :] = xj + coef * v
    pre = jnp.dot(xmod_ref[...], w_ref[...],
                  preferred_element_type=jnp.float32) + b_ref[...]
    out = jax.nn.sigmoid(pre)
    out1_ref[...] = out[:, :C].reshape(BB, S, C)
    out2_ref[...] = out[:, C:2 * C].reshape(BB, S, C)


@jax.jit
def kernel(x_lstm, position_s, position_e, W1, b1, W2, b2):
    w = jnp.concatenate([W1, W2], axis=1)
    b = jnp.concatenate([b1, b2]).reshape(1, 2 * C)
    pos_s = position_s.astype(jnp.int32)
    pos_e = position_e.astype(jnp.int32)
    grid = B // BB
    out1, out2 = pl.pallas_call(
        _body,
        grid_spec=pltpu.PrefetchScalarGridSpec(
            num_scalar_prefetch=2,
            grid=(grid,),
            in_specs=[
                pl.BlockSpec((BB, S, D), lambda g, *_: (g, 0, 0)),
                pl.BlockSpec((D, 2 * C), lambda g, *_: (0, 0)),
                pl.BlockSpec((1, 2 * C), lambda g, *_: (0, 0)),
            ],
            out_specs=[
                pl.BlockSpec((BB, S, C), lambda g, *_: (g, 0, 0)),
                pl.BlockSpec((BB, S, C), lambda g, *_: (g, 0, 0)),
            ],
            scratch_shapes=[pltpu.VMEM((BB * S, D), jnp.float32)],
        ),
        out_shape=[
            jax.ShapeDtypeStruct((B, S, C), jnp.float32),
            jax.ShapeDtypeStruct((B, S, C), jnp.float32),
        ],
        compiler_params=pltpu.CompilerParams(
            dimension_semantics=("arbitrary",),
        ),
    )(pos_s, pos_e, x_lstm, w, b)
    return (out1, out2)


# speedup vs baseline: 2.4527x; 1.4486x over previous
import jax
import jax.numpy as jnp
from jax.experimental import pallas as pl
from jax.experimental.pallas import tpu as pltpu

B, S, D, C = 1024, 200, 128, 49
BB = 32


def _body(s_ref, e_ref, x_ref, w1_ref, b1_ref, w2_ref, b2_ref,
          out1_ref, out2_ref, xmod_ref):
    g = pl.program_id(0)
    iota = jax.lax.broadcasted_iota(jnp.int32, (S, 1), 0)
    for j in range(BB):
        i = g * BB + j
        s = s_ref[i]
        e = e_ref[i]
        xj = x_ref[pl.ds(j * S, S), :]
        vs = x_ref[pl.ds(j * S + s, 1), :]
        ve = x_ref[pl.ds(j * S + e, 1), :]
        v = 0.5 * (vs + ve)
        coef = ((iota == s) | (iota == e)).astype(jnp.float32)
        xmod_ref[pl.ds(j * S, S), :] = xj + coef * v
    xmod = xmod_ref[...]
    out1_ref[...] = jax.nn.sigmoid(
        jnp.dot(xmod, w1_ref[...], preferred_element_type=jnp.float32)
        + b1_ref[...])
    out2_ref[...] = jax.nn.sigmoid(
        jnp.dot(xmod, w2_ref[...], preferred_element_type=jnp.float32)
        + b2_ref[...])


@jax.jit
def kernel(x_lstm, position_s, position_e, W1, b1, W2, b2):
    x2 = x_lstm.reshape(B * S, D)
    b1r = b1.reshape(1, C)
    b2r = b2.reshape(1, C)
    pos_s = position_s.astype(jnp.int32)
    pos_e = position_e.astype(jnp.int32)
    grid = B // BB
    out1, out2 = pl.pallas_call(
        _body,
        grid_spec=pltpu.PrefetchScalarGridSpec(
            num_scalar_prefetch=2,
            grid=(grid,),
            in_specs=[
                pl.BlockSpec((BB * S, D), lambda g, *_: (g, 0)),
                pl.BlockSpec((D, C), lambda g, *_: (0, 0)),
                pl.BlockSpec((1, C), lambda g, *_: (0, 0)),
                pl.BlockSpec((D, C), lambda g, *_: (0, 0)),
                pl.BlockSpec((1, C), lambda g, *_: (0, 0)),
            ],
            out_specs=[
                pl.BlockSpec((BB * S, C), lambda g, *_: (g, 0)),
                pl.BlockSpec((BB * S, C), lambda g, *_: (g, 0)),
            ],
            scratch_shapes=[pltpu.VMEM((BB * S, D), jnp.float32)],
        ),
        out_shape=[
            jax.ShapeDtypeStruct((B * S, C), jnp.float32),
            jax.ShapeDtypeStruct((B * S, C), jnp.float32),
        ],
        compiler_params=pltpu.CompilerParams(
            dimension_semantics=("arbitrary",),
        ),
    )(pos_s, pos_e, x2, W1, b1r, W2, b2r)
    return (out1.reshape(B, S, C), out2.reshape(B, S, C))
